# baseline (speedup 1.0000x reference)
import functools

import jax
import jax.numpy as jnp
from jax import lax
from jax.experimental import pallas as pl
from jax.experimental.pallas import tpu as pltpu
from jax.experimental.pallas import tpu_sc as plsc

EMBED_DIM = 64
BATCH = 16384
LANES = 16
NUM_CORES = 2
NUM_SUBCORES = 16
NW = NUM_CORES * NUM_SUBCORES
B_PER_W = BATCH // NW
STEP = LANES
NSTEP = B_PER_W // STEP


def _sc_body(src_hbm, tgt_hbm, emb_hbm, gat_hbm, out_hbm,
             sidx, tidx, ts_t, ts_s, tg_t, tg_s, outv, sem):
    c = lax.axis_index("c")
    s = lax.axis_index("s")
    wid = s * NUM_CORES + c
    base = wid * B_PER_W
    lane = lax.iota(jnp.int32, LANES)

    pltpu.sync_copy(src_hbm.at[pl.ds(base, B_PER_W)], sidx)
    pltpu.sync_copy(tgt_hbm.at[pl.ds(base, B_PER_W)], tidx)

    def step_body(st, carry):
        iv_t = tidx[pl.ds(st * STEP, STEP)]
        iv_s = sidx[pl.ds(st * STEP, STEP)]
        cps = []
        for j in range(STEP):
            m = lane == j
            it = jnp.sum(jnp.where(m, iv_t, 0))
            isrc = jnp.sum(jnp.where(m, iv_s, 0))
            cps.append(pltpu.async_copy(emb_hbm.at[it], ts_t.at[j], sem))
            cps.append(pltpu.async_copy(emb_hbm.at[isrc], ts_s.at[j], sem))
            cps.append(pltpu.async_copy(gat_hbm.at[it], tg_t.at[j], sem))
            cps.append(pltpu.async_copy(gat_hbm.at[isrc], tg_s.at[j], sem))
        for cp in cps:
            cp.wait()

        acc = jnp.zeros((LANES,), jnp.float32)
        for j in range(STEP):
            p = jnp.zeros((LANES,), jnp.float32)
            for k in range(EMBED_DIM // LANES):
                sl = pl.ds(k * LANES, LANES)
                gv = tg_t[j, sl] + tg_s[j, sl]
                gate = 1.0 / (1.0 + jnp.exp(-gv))
                p = p + gate * (ts_t[j, sl] - ts_s[j, sl])
            acc = jnp.where(lane == j, jnp.sum(p), acc)
        outv[pl.ds(st * STEP, STEP)] = acc
        return carry

    lax.fori_loop(0, NSTEP, step_body, 0)
    pltpu.sync_copy(outv, out_hbm.at[pl.ds(base, B_PER_W)])


@jax.jit
def kernel(source_nodes, target_nodes, node_embeddings, gates):
    mesh = plsc.VectorSubcoreMesh(core_axis_name="c", subcore_axis_name="s")
    k = pl.kernel(
        _sc_body,
        mesh=mesh,
        compiler_params=pltpu.CompilerParams(needs_layout_passes=False),
        out_type=jax.ShapeDtypeStruct((BATCH,), jnp.float32),
        scratch_types=[
            pltpu.VMEM((B_PER_W,), jnp.int32),
            pltpu.VMEM((B_PER_W,), jnp.int32),
            pltpu.VMEM((STEP, EMBED_DIM), jnp.float32),
            pltpu.VMEM((STEP, EMBED_DIM), jnp.float32),
            pltpu.VMEM((STEP, EMBED_DIM), jnp.float32),
            pltpu.VMEM((STEP, EMBED_DIM), jnp.float32),
            pltpu.VMEM((B_PER_W,), jnp.float32),
            pltpu.SemaphoreType.DMA,
        ],
    )
    return k(
        jnp.asarray(source_nodes, jnp.int32),
        jnp.asarray(target_nodes, jnp.int32),
        node_embeddings,
        gates,
    )
